# Initial kernel scaffold; baseline (speedup 1.0000x reference)
import jax
import jax.numpy as jnp
from jax.experimental import pallas as pl


def kernel(x_nchw, weight, bias, gamma, beta):
    raise NotImplementedError("write your pallas kernel here")


# speedup vs baseline: 3.1351x; 3.1351x over previous
import functools

import jax
import jax.numpy as jnp
import numpy as np
from jax.experimental import pallas as pl
from jax.experimental.pallas import tpu as pltpu

KH, KW = 3, 3
BN_EPS = 1e-5
BBLK1 = 16
BBLK2 = 16


def _conv_stats_kernel(x_ref, w_ref, b_ref, sel_ref, y_ref, stats_ref):
    BB, H, WC = x_ref.shape
    M = w_ref.shape[2]
    x = x_ref[...]
    zrow = jnp.zeros((BB, 1, WC), x.dtype)
    xu = jnp.concatenate([x[:, 1:, :], zrow], axis=1)
    xd = jnp.concatenate([zrow, x[:, :H - 1, :]], axis=1)
    R = BB * H
    acc = jnp.dot(x.reshape(R, WC), w_ref[1], preferred_element_type=jnp.float32)
    acc = acc + jnp.dot(xu.reshape(R, WC), w_ref[0],
                        preferred_element_type=jnp.float32)
    acc = acc + jnp.dot(xd.reshape(R, WC), w_ref[2],
                        preferred_element_type=jnp.float32)
    y = acc + b_ref[...]
    y_ref[...] = y.reshape(BB, H, M).astype(y_ref.dtype)

    ysum = jnp.sum(y, axis=0, keepdims=True)
    ysq = jnp.sum(y * y, axis=0, keepdims=True)
    st = jnp.concatenate([ysum, ysq], axis=0)
    stats_ref[...] = jnp.dot(st, sel_ref[...],
                             preferred_element_type=jnp.float32,
                             precision=jax.lax.Precision.HIGHEST)


def _bn_elu_kernel(y_ref, scale_ref, shift_ref, o_ref):
    v = y_ref[...].astype(jnp.float32) * scale_ref[...] + shift_ref[...]
    o_ref[...] = jnp.where(v > 0, v, jnp.exp(jnp.minimum(v, 0.0)) - 1.0)


def _band_weights(weight, W, Wo):
    Cin, Cout = weight.shape[0], weight.shape[1]
    w_t = jnp.transpose(weight, (2, 3, 0, 1)).astype(jnp.float32)
    P = np.zeros((KW, W, Wo), np.float32)
    iw = np.arange(W)
    for kw in range(KW):
        P[kw, iw, 2 * iw + kw] = 1.0
    band = jnp.einsum('kwo,hkic->hwioc', jnp.asarray(P), w_t)
    return band.reshape(KH, W * Cin, Wo * Cout)


@jax.jit
def _forward(x_nchw, weight, bias, gamma, beta):
    B, Cin, H, W = x_nchw.shape
    Cout = weight.shape[1]
    Wo = 2 * W + 1
    M = Wo * Cout
    nB1 = B // BBLK1
    nB2 = B // BBLK2

    x_rows = (jnp.transpose(x_nchw, (0, 2, 3, 1))
              .reshape(B, H, W * Cin).astype(jnp.bfloat16))
    w_band = _band_weights(weight, W, Wo).astype(jnp.bfloat16)
    bias_m = jnp.tile(bias.astype(jnp.float32), Wo).reshape(1, M)
    sel = jnp.tile(jnp.eye(Cout, dtype=jnp.float32), (Wo, 1))

    y_flat, stats = pl.pallas_call(
        _conv_stats_kernel,
        out_shape=(jax.ShapeDtypeStruct((B, H, M), jnp.bfloat16),
                   jax.ShapeDtypeStruct((nB1, 2, Cout), jnp.float32)),
        grid=(nB1,),
        in_specs=[
            pl.BlockSpec((BBLK1, H, W * Cin), lambda b: (b, 0, 0)),
            pl.BlockSpec((KH, W * Cin, M), lambda b: (0, 0, 0)),
            pl.BlockSpec((1, M), lambda b: (0, 0)),
            pl.BlockSpec((M, Cout), lambda b: (0, 0)),
        ],
        out_specs=(
            pl.BlockSpec((BBLK1, H, M), lambda b: (b, 0, 0)),
            pl.BlockSpec((None, 2, Cout), lambda b: (b, 0, 0)),
        ),
        compiler_params=pltpu.CompilerParams(dimension_semantics=("parallel",)),
    )(x_rows, w_band, bias_m, sel)

    cnt = float(B * H * Wo)
    s = jnp.sum(stats[:, 0, :], axis=0)
    ss = jnp.sum(stats[:, 1, :], axis=0)
    mean = s / cnt
    var = ss / cnt - mean * mean
    inv = jax.lax.rsqrt(var + BN_EPS)
    scale = gamma.astype(jnp.float32) * inv
    shift = beta.astype(jnp.float32) - mean * scale
    scale_m = jnp.tile(scale, Wo).reshape(1, M)
    shift_m = jnp.tile(shift, Wo).reshape(1, M)

    out_flat = pl.pallas_call(
        _bn_elu_kernel,
        out_shape=jax.ShapeDtypeStruct((B, H, M), jnp.float32),
        grid=(nB2,),
        in_specs=[
            pl.BlockSpec((BBLK2, H, M), lambda b: (b, 0, 0)),
            pl.BlockSpec((1, M), lambda b: (0, 0)),
            pl.BlockSpec((1, M), lambda b: (0, 0)),
        ],
        out_specs=pl.BlockSpec((BBLK2, H, M), lambda b: (b, 0, 0)),
        compiler_params=pltpu.CompilerParams(dimension_semantics=("parallel",)),
    )(y_flat, scale_m, shift_m)

    return jnp.transpose(out_flat.reshape(B, H, Wo, Cout), (0, 3, 1, 2))


def kernel(x_nchw, weight, bias, gamma, beta):
    return _forward(x_nchw, weight, bias, gamma, beta)
